# baseline (speedup 1.0000x reference)
import functools

import jax
import jax.numpy as jnp
from jax import lax
from jax.experimental import pallas as pl
from jax.experimental.pallas import tpu as pltpu
from jax.experimental.pallas import tpu_sc as plsc

WORD = 256
EMB = 16
H = WORD * EMB
T = 512
NB = H // T


def _make_sc_gather():
    info = plsc.get_sparse_core_info()
    nc, ns = info.num_cores, info.num_subcores
    nw = nc * ns
    n_active = 16
    rows_per = WORD // n_active
    mesh = plsc.VectorSubcoreMesh(core_axis_name="c", subcore_axis_name="s")

    @functools.partial(
        pl.kernel,
        mesh=mesh,
        compiler_params=pltpu.CompilerParams(needs_layout_passes=False),
        out_type=jax.ShapeDtypeStruct((WORD * EMB,), jnp.float32),
        scratch_types=[
            pltpu.VMEM((WORD * EMB,), jnp.float32),
            pltpu.VMEM((rows_per,), jnp.int32),
            pltpu.VMEM((rows_per * EMB,), jnp.float32),
        ],
    )
    def sc_gather(table_hbm, idx_hbm, out_hbm, table_v, idx_v, rows_v):
        wid = lax.axis_index("s") * nc + lax.axis_index("c")

        @pl.when(wid < n_active)
        def _():
            pltpu.sync_copy(table_hbm, table_v)
            pltpu.sync_copy(idx_hbm.at[pl.ds(wid * rows_per, rows_per)], idx_v)
            lanes = lax.iota(jnp.int32, 16)
            iv = idx_v[...]
            for k in range(rows_per):
                row = iv[k]
                vals = plsc.load_gather(table_v, [row * EMB + lanes])
                rows_v[pl.ds(k * EMB, EMB)] = vals
            pltpu.sync_copy(
                rows_v, out_hbm.at[pl.ds(wid * rows_per * EMB, rows_per * EMB)])

    return sc_gather


_sc_gather = _make_sc_gather()


def _lstm_body(x_ref, wi_ref, wg_ref, wo_ref,
               bi_ih, bg_ih, bo_ih, bi_hh, bg_hh, bo_hh,
               h_ref, c_ref):
    x = x_ref[...]
    dn = (((1,), (1,)), ((), ()))
    gi = lax.dot_general(x, wi_ref[...], dn, preferred_element_type=jnp.float32) \
        + bi_ih[...] + bi_hh[...]
    gg = lax.dot_general(x, wg_ref[...], dn, preferred_element_type=jnp.float32) \
        + bg_ih[...] + bg_hh[...]
    go = lax.dot_general(x, wo_ref[...], dn, preferred_element_type=jnp.float32) \
        + bo_ih[...] + bo_hh[...]
    i = jax.nn.sigmoid(gi)
    g = jnp.tanh(gg)
    o = jax.nn.sigmoid(go)
    c = i * g
    h_ref[...] = o * jnp.tanh(c)
    c_ref[...] = c


def _lstm_pallas(x, W_ih, b_ih2, b_hh2):
    w_spec = lambda off: pl.BlockSpec((T, H), lambda j, off=off: (j + off, 0))
    b_spec = lambda off: pl.BlockSpec((1, T), lambda j, off=off: (0, j + off))
    in_specs = [
        pl.BlockSpec((1, H), lambda j: (0, 0)),
        w_spec(0), w_spec(2 * NB), w_spec(3 * NB),
        b_spec(0), b_spec(2 * NB), b_spec(3 * NB),
        b_spec(0), b_spec(2 * NB), b_spec(3 * NB),
    ]
    out_specs = [pl.BlockSpec((1, T), lambda j: (0, j))] * 2
    out_shape = [jax.ShapeDtypeStruct((1, H), jnp.float32)] * 2
    return pl.pallas_call(
        _lstm_body,
        grid=(NB,),
        in_specs=in_specs,
        out_specs=out_specs,
        out_shape=out_shape,
    )(x, W_ih, W_ih, W_ih, b_ih2, b_ih2, b_ih2, b_hh2, b_hh2, b_hh2)


def kernel(input, table, W_ih, W_hh, b_ih, b_hh):
    del W_hh
    idx = input.astype(jnp.int32)
    emb = _sc_gather(table.reshape(WORD * EMB), idx)
    x = emb.reshape(1, H)
    h, c = _lstm_pallas(x, W_ih,
                        b_ih.reshape(1, 4 * H), b_hh.reshape(1, 4 * H))
    out = h.reshape(1, 1, H)
    return (out, out, c.reshape(1, 1, H))


# speedup vs baseline: 1.0398x; 1.0398x over previous
import functools

import jax
import jax.numpy as jnp
from jax import lax
from jax.experimental import pallas as pl
from jax.experimental.pallas import tpu as pltpu
from jax.experimental.pallas import tpu_sc as plsc

WORD = 256
EMB = 16
H = WORD * EMB
T = 128
NB = H // T


def _make_sc_gather():
    info = plsc.get_sparse_core_info()
    nc, ns = info.num_cores, info.num_subcores
    nw = nc * ns
    n_active = 16
    rows_per = WORD // n_active
    mesh = plsc.VectorSubcoreMesh(core_axis_name="c", subcore_axis_name="s")

    @functools.partial(
        pl.kernel,
        mesh=mesh,
        compiler_params=pltpu.CompilerParams(needs_layout_passes=False),
        out_type=jax.ShapeDtypeStruct((WORD * EMB,), jnp.float32),
        scratch_types=[
            pltpu.VMEM((WORD * EMB,), jnp.float32),
            pltpu.VMEM((rows_per,), jnp.int32),
            pltpu.VMEM((rows_per * EMB,), jnp.float32),
        ],
    )
    def sc_gather(table_hbm, idx_hbm, out_hbm, table_v, idx_v, rows_v):
        wid = lax.axis_index("s") * nc + lax.axis_index("c")

        @pl.when(wid < n_active)
        def _():
            pltpu.sync_copy(table_hbm, table_v)
            pltpu.sync_copy(idx_hbm.at[pl.ds(wid * rows_per, rows_per)], idx_v)
            lanes = lax.iota(jnp.int32, 16)
            iv = idx_v[...]
            for k in range(rows_per):
                row = iv[k]
                vals = plsc.load_gather(table_v, [row * EMB + lanes])
                rows_v[pl.ds(k * EMB, EMB)] = vals
            pltpu.sync_copy(
                rows_v, out_hbm.at[pl.ds(wid * rows_per * EMB, rows_per * EMB)])

    return sc_gather


_sc_gather = _make_sc_gather()


def _lstm_body(x_ref, wi_ref, wg_ref, wo_ref,
               bi_ih, bg_ih, bo_ih, bi_hh, bg_hh, bo_hh,
               h_ref, c_ref):
    x = x_ref[...]
    dn = (((1,), (1,)), ((), ()))
    gi = lax.dot_general(x, wi_ref[...], dn, preferred_element_type=jnp.float32) \
        + bi_ih[...] + bi_hh[...]
    gg = lax.dot_general(x, wg_ref[...], dn, preferred_element_type=jnp.float32) \
        + bg_ih[...] + bg_hh[...]
    go = lax.dot_general(x, wo_ref[...], dn, preferred_element_type=jnp.float32) \
        + bo_ih[...] + bo_hh[...]
    i = jax.nn.sigmoid(gi)
    g = jnp.tanh(gg)
    o = jax.nn.sigmoid(go)
    c = i * g
    h_ref[...] = o * jnp.tanh(c)
    c_ref[...] = c


def _lstm_pallas(x, W_ih, b_ih2, b_hh2):
    w_spec = lambda off: pl.BlockSpec((T, H), lambda j, off=off: (j + off, 0))
    b_spec = lambda off: pl.BlockSpec((1, T), lambda j, off=off: (0, j + off))
    in_specs = [
        pl.BlockSpec((1, H), lambda j: (0, 0)),
        w_spec(0), w_spec(2 * NB), w_spec(3 * NB),
        b_spec(0), b_spec(2 * NB), b_spec(3 * NB),
        b_spec(0), b_spec(2 * NB), b_spec(3 * NB),
    ]
    out_specs = [pl.BlockSpec((1, T), lambda j: (0, j))] * 2
    out_shape = [jax.ShapeDtypeStruct((1, H), jnp.float32)] * 2
    return pl.pallas_call(
        _lstm_body,
        grid=(NB,),
        in_specs=in_specs,
        out_specs=out_specs,
        out_shape=out_shape,
    )(x, W_ih, W_ih, W_ih, b_ih2, b_ih2, b_ih2, b_hh2, b_hh2, b_hh2)


def kernel(input, table, W_ih, W_hh, b_ih, b_hh):
    del W_hh
    idx = input.astype(jnp.int32)
    emb = _sc_gather(table.reshape(WORD * EMB), idx)
    x = emb.reshape(1, H)
    h, c = _lstm_pallas(x, W_ih,
                        b_ih.reshape(1, 4 * H), b_hh.reshape(1, 4 * H))
    out = h.reshape(1, 1, H)
    return (out, out, c.reshape(1, 1, H))


# speedup vs baseline: 1.0847x; 1.0432x over previous
import functools

import jax
import jax.numpy as jnp
from jax import lax
from jax.experimental import pallas as pl
from jax.experimental.pallas import tpu as pltpu
from jax.experimental.pallas import tpu_sc as plsc

WORD = 256
EMB = 16
H = WORD * EMB
T = 256
NB = H // T


def _make_sc_gather():
    info = plsc.get_sparse_core_info()
    nc, ns = info.num_cores, info.num_subcores
    nw = nc * ns
    n_active = 16
    rows_per = WORD // n_active
    mesh = plsc.VectorSubcoreMesh(core_axis_name="c", subcore_axis_name="s")

    @functools.partial(
        pl.kernel,
        mesh=mesh,
        compiler_params=pltpu.CompilerParams(needs_layout_passes=False),
        out_type=jax.ShapeDtypeStruct((WORD * EMB,), jnp.float32),
        scratch_types=[
            pltpu.VMEM((WORD * EMB,), jnp.float32),
            pltpu.VMEM((rows_per,), jnp.int32),
            pltpu.VMEM((rows_per * EMB,), jnp.float32),
        ],
    )
    def sc_gather(table_hbm, idx_hbm, out_hbm, table_v, idx_v, rows_v):
        wid = lax.axis_index("s") * nc + lax.axis_index("c")

        @pl.when(wid < n_active)
        def _():
            pltpu.sync_copy(table_hbm, table_v)
            pltpu.sync_copy(idx_hbm.at[pl.ds(wid * rows_per, rows_per)], idx_v)
            lanes = lax.iota(jnp.int32, 16)
            iv = idx_v[...]
            for k in range(rows_per):
                row = iv[k]
                vals = plsc.load_gather(table_v, [row * EMB + lanes])
                rows_v[pl.ds(k * EMB, EMB)] = vals
            pltpu.sync_copy(
                rows_v, out_hbm.at[pl.ds(wid * rows_per * EMB, rows_per * EMB)])

    return sc_gather


_sc_gather = _make_sc_gather()


KS = 2
HK = H // KS


def _lstm_body(*refs):
    x_ref = refs[0]
    w_refs = refs[1:1 + 3 * KS]
    bi_ih, bg_ih, bo_ih, bi_hh, bg_hh, bo_hh = refs[1 + 3 * KS:7 + 3 * KS]
    h_ref, c_ref = refs[7 + 3 * KS:]
    x = x_ref[...]
    dn = (((1,), (1,)), ((), ()))

    def mv(gate):
        acc = None
        for ks in range(KS):
            part = lax.dot_general(
                x[:, ks * HK:(ks + 1) * HK], w_refs[gate * KS + ks][...],
                dn, preferred_element_type=jnp.float32)
            acc = part if acc is None else acc + part
        return acc

    gi = mv(0) + bi_ih[...] + bi_hh[...]
    gg = mv(1) + bg_ih[...] + bg_hh[...]
    go = mv(2) + bo_ih[...] + bo_hh[...]
    i = jax.nn.sigmoid(gi)
    g = jnp.tanh(gg)
    o = jax.nn.sigmoid(go)
    c = i * g
    h_ref[...] = o * jnp.tanh(c)
    c_ref[...] = c


def _lstm_pallas(x, W_ih, b_ih2, b_hh2):
    w_spec = lambda off, ks: pl.BlockSpec(
        (T, HK), lambda j, off=off, ks=ks: (j + off, ks))
    b_spec = lambda off: pl.BlockSpec((1, T), lambda j, off=off: (0, j + off))
    in_specs = [pl.BlockSpec((1, H), lambda j: (0, 0))]
    in_specs += [w_spec(off, ks)
                 for off in (0, 2 * NB, 3 * NB) for ks in range(KS)]
    in_specs += [b_spec(0), b_spec(2 * NB), b_spec(3 * NB)] * 2
    out_specs = [pl.BlockSpec((1, T), lambda j: (0, j))] * 2
    out_shape = [jax.ShapeDtypeStruct((1, H), jnp.float32)] * 2
    return pl.pallas_call(
        _lstm_body,
        grid=(NB,),
        in_specs=in_specs,
        out_specs=out_specs,
        out_shape=out_shape,
    )(x, *([W_ih] * (3 * KS)),
      b_ih2, b_ih2, b_ih2, b_hh2, b_hh2, b_hh2)


def kernel(input, table, W_ih, W_hh, b_ih, b_hh):
    del W_hh
    idx = input.astype(jnp.int32)
    emb = _sc_gather(table.reshape(WORD * EMB), idx)
    x = emb.reshape(1, H)
    h, c = _lstm_pallas(x, W_ih,
                        b_ih.reshape(1, 4 * H), b_hh.reshape(1, 4 * H))
    out = h.reshape(1, 1, H)
    return (out, out, c.reshape(1, 1, H))


# speedup vs baseline: 1.0897x; 1.0046x over previous
import functools

import jax
import jax.numpy as jnp
from jax import lax
from jax.experimental import pallas as pl
from jax.experimental.pallas import tpu as pltpu
from jax.experimental.pallas import tpu_sc as plsc

WORD = 256
EMB = 16
H = WORD * EMB
T = 256
NB = H // T


def _make_sc_gather():
    info = plsc.get_sparse_core_info()
    nc, ns = info.num_cores, info.num_subcores
    nw = nc * ns
    n_active = 16
    rows_per = WORD // n_active
    mesh = plsc.VectorSubcoreMesh(core_axis_name="c", subcore_axis_name="s")

    @functools.partial(
        pl.kernel,
        mesh=mesh,
        compiler_params=pltpu.CompilerParams(needs_layout_passes=False),
        out_type=jax.ShapeDtypeStruct((WORD * EMB,), jnp.float32),
        scratch_types=[
            pltpu.VMEM((WORD * EMB,), jnp.float32),
            pltpu.VMEM((rows_per,), jnp.int32),
            pltpu.VMEM((rows_per * EMB,), jnp.float32),
        ],
    )
    def sc_gather(table_hbm, idx_hbm, out_hbm, table_v, idx_v, rows_v):
        wid = lax.axis_index("s") * nc + lax.axis_index("c")

        @pl.when(wid < n_active)
        def _():
            pltpu.sync_copy(table_hbm, table_v)
            pltpu.sync_copy(idx_hbm.at[pl.ds(wid * rows_per, rows_per)], idx_v)
            lanes = lax.iota(jnp.int32, 16)
            iv = idx_v[...]
            for k in range(rows_per):
                row = iv[k]
                vals = plsc.load_gather(table_v, [row * EMB + lanes])
                rows_v[pl.ds(k * EMB, EMB)] = vals
            pltpu.sync_copy(
                rows_v, out_hbm.at[pl.ds(wid * rows_per * EMB, rows_per * EMB)])

    return sc_gather


_sc_gather = _make_sc_gather()


KS = 4
HK = H // KS


def _lstm_body(*refs):
    x_ref = refs[0]
    w_refs = refs[1:1 + 3 * KS]
    bi_ih, bg_ih, bo_ih, bi_hh, bg_hh, bo_hh = refs[1 + 3 * KS:7 + 3 * KS]
    h_ref, c_ref = refs[7 + 3 * KS:]
    x = x_ref[...]
    dn = (((1,), (1,)), ((), ()))

    def mv(gate):
        acc = None
        for ks in range(KS):
            part = lax.dot_general(
                x[:, ks * HK:(ks + 1) * HK], w_refs[gate * KS + ks][...],
                dn, preferred_element_type=jnp.float32)
            acc = part if acc is None else acc + part
        return acc

    gi = mv(0) + bi_ih[...] + bi_hh[...]
    gg = mv(1) + bg_ih[...] + bg_hh[...]
    go = mv(2) + bo_ih[...] + bo_hh[...]
    i = jax.nn.sigmoid(gi)
    g = jnp.tanh(gg)
    o = jax.nn.sigmoid(go)
    c = i * g
    h_ref[...] = o * jnp.tanh(c)
    c_ref[...] = c


def _lstm_pallas(x, W_ih, b_ih2, b_hh2):
    w_spec = lambda off, ks: pl.BlockSpec(
        (T, HK), lambda j, off=off, ks=ks: (j + off, ks))
    b_spec = lambda off: pl.BlockSpec((1, T), lambda j, off=off: (0, j + off))
    in_specs = [pl.BlockSpec((1, H), lambda j: (0, 0))]
    in_specs += [w_spec(off, ks)
                 for off in (0, 2 * NB, 3 * NB) for ks in range(KS)]
    in_specs += [b_spec(0), b_spec(2 * NB), b_spec(3 * NB)] * 2
    out_specs = [pl.BlockSpec((1, T), lambda j: (0, j))] * 2
    out_shape = [jax.ShapeDtypeStruct((1, H), jnp.float32)] * 2
    return pl.pallas_call(
        _lstm_body,
        grid=(NB,),
        in_specs=in_specs,
        out_specs=out_specs,
        out_shape=out_shape,
    )(x, *([W_ih] * (3 * KS)),
      b_ih2, b_ih2, b_ih2, b_hh2, b_hh2, b_hh2)


def kernel(input, table, W_ih, W_hh, b_ih, b_hh):
    del W_hh
    idx = input.astype(jnp.int32)
    emb = _sc_gather(table.reshape(WORD * EMB), idx)
    x = emb.reshape(1, H)
    h, c = _lstm_pallas(x, W_ih,
                        b_ih.reshape(1, 4 * H), b_hh.reshape(1, 4 * H))
    out = h.reshape(1, 1, H)
    return (out, out, c.reshape(1, 1, H))


# speedup vs baseline: 1.1067x; 1.0157x over previous
import functools

import jax
import jax.numpy as jnp
from jax import lax
from jax.experimental import pallas as pl
from jax.experimental.pallas import tpu as pltpu
from jax.experimental.pallas import tpu_sc as plsc

WORD = 256
EMB = 16
H = WORD * EMB
T = 256
NB = H // T


def _make_sc_gather():
    info = plsc.get_sparse_core_info()
    nc, ns = info.num_cores, info.num_subcores
    nw = nc * ns
    n_active = 16
    rows_per = WORD // n_active
    mesh = plsc.VectorSubcoreMesh(core_axis_name="c", subcore_axis_name="s")

    @functools.partial(
        pl.kernel,
        mesh=mesh,
        compiler_params=pltpu.CompilerParams(needs_layout_passes=False),
        out_type=jax.ShapeDtypeStruct((WORD * EMB,), jnp.float32),
        scratch_types=[
            pltpu.VMEM((WORD * EMB,), jnp.float32),
            pltpu.VMEM((rows_per,), jnp.int32),
            pltpu.VMEM((rows_per * EMB,), jnp.float32),
        ],
    )
    def sc_gather(table_hbm, idx_hbm, out_hbm, table_v, idx_v, rows_v):
        wid = lax.axis_index("s") * nc + lax.axis_index("c")

        @pl.when(wid < n_active)
        def _():
            pltpu.sync_copy(table_hbm, table_v)
            pltpu.sync_copy(idx_hbm.at[pl.ds(wid * rows_per, rows_per)], idx_v)
            lanes = lax.iota(jnp.int32, 16)
            iv = idx_v[...]

            dnums = lax.GatherDimensionNumbers(
                offset_dims=(), collapsed_slice_dims=(0,),
                start_index_map=(0,))

            def body(k, carry):
                kvec = jnp.full((16,), 0, jnp.int32) + k
                row = lax.gather(
                    iv, kvec[:, None], dnums, (1,),
                    mode=lax.GatherScatterMode.PROMISE_IN_BOUNDS)
                vals = plsc.load_gather(table_v, [row * EMB + lanes])
                rows_v[pl.ds(k * EMB, EMB)] = vals
                return carry

            lax.fori_loop(0, rows_per, body, 0)
            pltpu.sync_copy(
                rows_v, out_hbm.at[pl.ds(wid * rows_per * EMB, rows_per * EMB)])

    return sc_gather


_sc_gather = _make_sc_gather()


KS = 4
HK = H // KS


def _lstm_body(*refs):
    x_ref = refs[0]
    w_refs = refs[1:1 + 3 * KS]
    bi_ih, bg_ih, bo_ih, bi_hh, bg_hh, bo_hh = refs[1 + 3 * KS:7 + 3 * KS]
    h1_ref, h2_ref, c_ref = refs[7 + 3 * KS:]
    x = x_ref[...]
    dn = (((1,), (1,)), ((), ()))

    def mv(gate):
        acc = None
        for ks in range(KS):
            part = lax.dot_general(
                x[:, ks * HK:(ks + 1) * HK], w_refs[gate * KS + ks][...],
                dn, preferred_element_type=jnp.float32)
            acc = part if acc is None else acc + part
        return acc

    gi = mv(0) + bi_ih[...] + bi_hh[...]
    gg = mv(1) + bg_ih[...] + bg_hh[...]
    go = mv(2) + bo_ih[...] + bo_hh[...]
    i = jax.nn.sigmoid(gi)
    g = jnp.tanh(gg)
    o = jax.nn.sigmoid(go)
    c = i * g
    h = o * jnp.tanh(c)
    h1_ref[...] = h
    h2_ref[...] = h
    c_ref[...] = c


def _lstm_pallas(x, W_ih, b_ih2, b_hh2):
    w_spec = lambda off, ks: pl.BlockSpec(
        (T, HK), lambda j, off=off, ks=ks: (j + off, ks))
    b_spec = lambda off: pl.BlockSpec((1, T), lambda j, off=off: (0, j + off))
    in_specs = [pl.BlockSpec((1, H), lambda j: (0, 0))]
    in_specs += [w_spec(off, ks)
                 for off in (0, 2 * NB, 3 * NB) for ks in range(KS)]
    in_specs += [b_spec(0), b_spec(2 * NB), b_spec(3 * NB)] * 2
    out_specs = [pl.BlockSpec((1, T), lambda j: (0, j))] * 3
    out_shape = [jax.ShapeDtypeStruct((1, H), jnp.float32)] * 3
    return pl.pallas_call(
        _lstm_body,
        grid=(NB,),
        in_specs=in_specs,
        out_specs=out_specs,
        out_shape=out_shape,
    )(x, *([W_ih] * (3 * KS)),
      b_ih2, b_ih2, b_ih2, b_hh2, b_hh2, b_hh2)


def kernel(input, table, W_ih, W_hh, b_ih, b_hh):
    del W_hh
    idx = input.astype(jnp.int32)
    emb = _sc_gather(table.reshape(WORD * EMB), idx)
    x = emb.reshape(1, H)
    h1, h2, c = _lstm_pallas(x, W_ih,
                             b_ih.reshape(1, 4 * H), b_hh.reshape(1, 4 * H))
    return (h1.reshape(1, 1, H), h2.reshape(1, 1, H), c.reshape(1, 1, H))


# speedup vs baseline: 1.1219x; 1.0137x over previous
import functools

import jax
import jax.numpy as jnp
from jax import lax
from jax.experimental import pallas as pl
from jax.experimental.pallas import tpu as pltpu
from jax.experimental.pallas import tpu_sc as plsc

WORD = 256
EMB = 16
H = WORD * EMB
T = 256
NB = H // T


def _make_sc_gather():
    info = plsc.get_sparse_core_info()
    nc, ns = info.num_cores, info.num_subcores
    nw = nc * ns
    n_active = 16
    rows_per = WORD // n_active
    mesh = plsc.VectorSubcoreMesh(core_axis_name="c", subcore_axis_name="s",
                                  num_cores=1)

    @functools.partial(
        pl.kernel,
        mesh=mesh,
        compiler_params=pltpu.CompilerParams(needs_layout_passes=False),
        out_type=jax.ShapeDtypeStruct((WORD * EMB,), jnp.float32),
        scratch_types=[
            pltpu.VMEM((WORD * EMB,), jnp.float32),
            pltpu.VMEM((rows_per,), jnp.int32),
            pltpu.VMEM((rows_per * EMB,), jnp.float32),
        ],
    )
    def sc_gather(table_hbm, idx_hbm, out_hbm, table_v, idx_v, rows_v):
        wid = lax.axis_index("s")

        @pl.when(wid < n_active)
        def _():
            pltpu.sync_copy(table_hbm, table_v)
            pltpu.sync_copy(idx_hbm.at[pl.ds(wid * rows_per, rows_per)], idx_v)
            lanes = lax.iota(jnp.int32, 16)
            iv = idx_v[...]

            dnums = lax.GatherDimensionNumbers(
                offset_dims=(), collapsed_slice_dims=(0,),
                start_index_map=(0,))

            def body(k, carry):
                kvec = jnp.full((16,), 0, jnp.int32) + k
                row = lax.gather(
                    iv, kvec[:, None], dnums, (1,),
                    mode=lax.GatherScatterMode.PROMISE_IN_BOUNDS)
                vals = plsc.load_gather(table_v, [row * EMB + lanes])
                rows_v[pl.ds(k * EMB, EMB)] = vals
                return carry

            lax.fori_loop(0, rows_per, body, 0)
            pltpu.sync_copy(
                rows_v, out_hbm.at[pl.ds(wid * rows_per * EMB, rows_per * EMB)])

    return sc_gather


_sc_gather = _make_sc_gather()


KS = 4
HK = H // KS


def _lstm_body(*refs):
    x_ref = refs[0]
    w_refs = refs[1:1 + 3 * KS]
    bi_ih, bg_ih, bo_ih, bi_hh, bg_hh, bo_hh = refs[1 + 3 * KS:7 + 3 * KS]
    h1_ref, h2_ref, c_ref = refs[7 + 3 * KS:]
    x = x_ref[...]
    dn = (((1,), (1,)), ((), ()))

    def mv(gate):
        acc = None
        for ks in range(KS):
            part = lax.dot_general(
                x[:, ks * HK:(ks + 1) * HK], w_refs[gate * KS + ks][...],
                dn, preferred_element_type=jnp.float32)
            acc = part if acc is None else acc + part
        return acc

    gi = mv(0) + bi_ih[...] + bi_hh[...]
    gg = mv(1) + bg_ih[...] + bg_hh[...]
    go = mv(2) + bo_ih[...] + bo_hh[...]
    i = jax.nn.sigmoid(gi)
    g = jnp.tanh(gg)
    o = jax.nn.sigmoid(go)
    c = i * g
    h = o * jnp.tanh(c)
    h1_ref[...] = h
    h2_ref[...] = h
    c_ref[...] = c


def _lstm_pallas(x, W_ih, b_ih2, b_hh2):
    w_spec = lambda off, ks: pl.BlockSpec(
        (T, HK), lambda j, off=off, ks=ks: (j + off, ks))
    b_spec = lambda off: pl.BlockSpec((1, T), lambda j, off=off: (0, j + off))
    in_specs = [pl.BlockSpec((1, H), lambda j: (0, 0))]
    in_specs += [w_spec(off, ks)
                 for off in (0, 2 * NB, 3 * NB) for ks in range(KS)]
    in_specs += [b_spec(0), b_spec(2 * NB), b_spec(3 * NB)] * 2
    out_specs = [pl.BlockSpec((1, T), lambda j: (0, j))] * 3
    out_shape = [jax.ShapeDtypeStruct((1, H), jnp.float32)] * 3
    return pl.pallas_call(
        _lstm_body,
        grid=(NB,),
        in_specs=in_specs,
        out_specs=out_specs,
        out_shape=out_shape,
    )(x, *([W_ih] * (3 * KS)),
      b_ih2, b_ih2, b_ih2, b_hh2, b_hh2, b_hh2)


def kernel(input, table, W_ih, W_hh, b_ih, b_hh):
    del W_hh
    idx = input.astype(jnp.int32)
    emb = _sc_gather(table.reshape(WORD * EMB), idx)
    x = emb.reshape(1, H)
    h1, h2, c = _lstm_pallas(x, W_ih,
                             b_ih.reshape(1, 4 * H), b_hh.reshape(1, 4 * H))
    return (h1.reshape(1, 1, H), h2.reshape(1, 1, H), c.reshape(1, 1, H))


# speedup vs baseline: 1.3496x; 1.2030x over previous
import jax
import jax.numpy as jnp
from jax import lax
from jax.experimental import pallas as pl
from jax.experimental.pallas import tpu as pltpu

WORD = 256
EMB = 16
H = WORD * EMB
T = 256
NB = H // T
KS = 4
HK = H // KS


def _gather_body(idx_ref, table_ref, emb_ref):
    v_iota = lax.broadcasted_iota(jnp.int32, (WORD, WORD), 1)
    onehot = (v_iota == idx_ref[...]).astype(jnp.float32)
    emb_ref[...] = lax.dot_general(
        onehot, table_ref[...], (((1,), (0,)), ((), ())),
        preferred_element_type=jnp.float32)


def _gather_pallas(idx2d, table):
    return pl.pallas_call(
        _gather_body,
        out_shape=jax.ShapeDtypeStruct((WORD, EMB), jnp.float32),
    )(idx2d, table)


def _lstm_body(*refs):
    x_ref = refs[0]
    w_refs = refs[1:1 + 3 * KS]
    bi_ih, bg_ih, bo_ih, bi_hh, bg_hh, bo_hh = refs[1 + 3 * KS:7 + 3 * KS]
    h1_ref, h2_ref, c_ref = refs[7 + 3 * KS:]
    x = x_ref[...]
    dn = (((1,), (1,)), ((), ()))

    def mv(gate):
        acc = None
        for ks in range(KS):
            part = lax.dot_general(
                x[:, ks * HK:(ks + 1) * HK], w_refs[gate * KS + ks][...],
                dn, preferred_element_type=jnp.float32)
            acc = part if acc is None else acc + part
        return acc

    gi = mv(0) + bi_ih[...] + bi_hh[...]
    gg = mv(1) + bg_ih[...] + bg_hh[...]
    go = mv(2) + bo_ih[...] + bo_hh[...]
    i = jax.nn.sigmoid(gi)
    g = jnp.tanh(gg)
    o = jax.nn.sigmoid(go)
    c = i * g
    h = o * jnp.tanh(c)
    h1_ref[...] = h
    h2_ref[...] = h
    c_ref[...] = c


def _lstm_pallas(x, W_ih, b_ih2, b_hh2):
    w_spec = lambda off, ks: pl.BlockSpec(
        (T, HK), lambda j, off=off, ks=ks: (j + off, ks))
    b_spec = lambda off: pl.BlockSpec((1, T), lambda j, off=off: (0, j + off))
    in_specs = [pl.BlockSpec((1, H), lambda j: (0, 0))]
    in_specs += [w_spec(off, ks)
                 for off in (0, 2 * NB, 3 * NB) for ks in range(KS)]
    in_specs += [b_spec(0), b_spec(2 * NB), b_spec(3 * NB)] * 2
    out_specs = [pl.BlockSpec((1, T), lambda j: (0, j))] * 3
    out_shape = [jax.ShapeDtypeStruct((1, H), jnp.float32)] * 3
    return pl.pallas_call(
        _lstm_body,
        grid=(NB,),
        in_specs=in_specs,
        out_specs=out_specs,
        out_shape=out_shape,
    )(x, *([W_ih] * (3 * KS)),
      b_ih2, b_ih2, b_ih2, b_hh2, b_hh2, b_hh2)


def kernel(input, table, W_ih, W_hh, b_ih, b_hh):
    del W_hh
    idx2d = input.astype(jnp.int32).reshape(WORD, 1)
    emb = _gather_pallas(idx2d, table)
    x = emb.reshape(1, H)
    h1, h2, c = _lstm_pallas(x, W_ih,
                             b_ih.reshape(1, 4 * H), b_hh.reshape(1, 4 * H))
    return (h1.reshape(1, 1, H), h2.reshape(1, 1, H), c.reshape(1, 1, H))


# speedup vs baseline: 1.3792x; 1.0219x over previous
import jax
import jax.numpy as jnp
from jax import lax
from jax.experimental import pallas as pl

WORD = 256
EMB = 16
H = WORD * EMB
T = 256
NB = H // T
KS = 8
HK = H // KS


def _gather_body(idx_ref, table_ref, emb_ref):
    v_iota = lax.broadcasted_iota(jnp.int32, (WORD, WORD), 0)
    onehot_t = (v_iota == idx_ref[...][None, :]).astype(jnp.float32)
    emb_ref[...] = lax.dot_general(
        onehot_t, table_ref[...], (((0,), (0,)), ((), ())),
        preferred_element_type=jnp.float32)


def _gather_pallas(idx1d, table):
    return pl.pallas_call(
        _gather_body,
        out_shape=jax.ShapeDtypeStruct((WORD, EMB), jnp.float32),
    )(idx1d, table)


def _lstm_body(*refs):
    x_ref = refs[0]
    w_refs = refs[1:1 + 3 * KS]
    bi_ih, bg_ih, bo_ih, bi_hh, bg_hh, bo_hh = refs[1 + 3 * KS:7 + 3 * KS]
    h1_ref, h2_ref, c_ref = refs[7 + 3 * KS:]
    x = x_ref[...]
    dn = (((1,), (1,)), ((), ()))

    def bias(ref):
        return ref[...].reshape(1, T)

    def mv(gate):
        acc = None
        for ks in range(KS):
            part = lax.dot_general(
                x[:, ks * HK:(ks + 1) * HK], w_refs[gate * KS + ks][...],
                dn, preferred_element_type=jnp.float32)
            acc = part if acc is None else acc + part
        return acc

    gi = mv(0) + bias(bi_ih) + bias(bi_hh)
    gg = mv(1) + bias(bg_ih) + bias(bg_hh)
    go = mv(2) + bias(bo_ih) + bias(bo_hh)
    i = jax.nn.sigmoid(gi)
    g = jnp.tanh(gg)
    o = jax.nn.sigmoid(go)
    c = i * g
    h = o * jnp.tanh(c)
    h1_ref[...] = h
    h2_ref[...] = h
    c_ref[...] = c


def _lstm_pallas(x, W_ih, b_ih2, b_hh2):
    w_spec = lambda off, ks: pl.BlockSpec(
        (T, HK), lambda j, off=off, ks=ks: (j + off, ks))
    b_spec = lambda off: pl.BlockSpec((T,), lambda j, off=off: (j + off,))
    in_specs = [pl.BlockSpec((1, H), lambda j: (0, 0))]
    in_specs += [w_spec(off, ks)
                 for off in (0, 2 * NB, 3 * NB) for ks in range(KS)]
    in_specs += [b_spec(0), b_spec(2 * NB), b_spec(3 * NB)] * 2
    out_specs = [pl.BlockSpec((1, T), lambda j: (0, j))] * 3
    out_shape = [jax.ShapeDtypeStruct((1, H), jnp.float32)] * 3
    return pl.pallas_call(
        _lstm_body,
        grid=(NB,),
        in_specs=in_specs,
        out_specs=out_specs,
        out_shape=out_shape,
    )(x, *([W_ih] * (3 * KS)),
      b_ih2, b_ih2, b_ih2, b_hh2, b_hh2, b_hh2)


def kernel(input, table, W_ih, W_hh, b_ih, b_hh):
    del W_hh
    emb = _gather_pallas(input.astype(jnp.int32), table)
    x = emb.reshape(1, H)
    h1, h2, c = _lstm_pallas(x, W_ih, b_ih, b_hh)
    return (h1.reshape(1, 1, H), h2.reshape(1, 1, H), c.reshape(1, 1, H))


# speedup vs baseline: 1.4426x; 1.0459x over previous
import jax
import jax.numpy as jnp
from jax import lax
from jax.experimental import pallas as pl
from jax.experimental.pallas import tpu as pltpu

WORD = 256
EMB = 16
H = WORD * EMB
T = 256
NB = H // T
KS = 8
HK = H // KS


WPC = HK // EMB


def _lstm_body(*refs):
    idx_ref, table_ref = refs[0], refs[1]
    w_refs = refs[2:2 + 3 * KS]
    bi_ih, bg_ih, bo_ih, bi_hh, bg_hh, bo_hh = refs[2 + 3 * KS:8 + 3 * KS]
    h1_ref, h2_ref, c_ref, x_s = refs[8 + 3 * KS:]
    j = pl.program_id(0)

    @pl.when(j == 0)
    def _():
        v_iota = lax.broadcasted_iota(jnp.int32, (WORD, WORD), 0)
        onehot_t = (v_iota == idx_ref[...][None, :]).astype(jnp.float32)
        emb = lax.dot_general(
            onehot_t, table_ref[...], (((0,), (0,)), ((), ())),
            preferred_element_type=jnp.float32)
        sel = (lax.broadcasted_iota(jnp.int32, (EMB, HK), 0)
               == lax.broadcasted_iota(jnp.int32, (EMB, HK), 1) % EMB
               ).astype(jnp.float32)
        mask = (lax.broadcasted_iota(jnp.int32, (WPC, HK), 0)
                == lax.broadcasted_iota(jnp.int32, (WPC, HK), 1) // EMB
                ).astype(jnp.float32)
        ones = jnp.ones((1, WPC), jnp.float32)
        for ks in range(KS):
            g = lax.dot_general(
                emb[ks * WPC:(ks + 1) * WPC, :], sel,
                (((1,), (0,)), ((), ())),
                preferred_element_type=jnp.float32)
            xc = lax.dot_general(
                ones, g * mask, (((1,), (0,)), ((), ())),
                preferred_element_type=jnp.float32)
            x_s[0, ks * HK:(ks + 1) * HK] = xc[0, :]

    x = x_s[...]
    dn = (((1,), (1,)), ((), ()))

    def bias(ref):
        return ref[...].reshape(1, T)

    def mv(gate):
        acc = None
        for ks in range(KS):
            part = lax.dot_general(
                x[:, ks * HK:(ks + 1) * HK], w_refs[gate * KS + ks][...],
                dn, preferred_element_type=jnp.float32)
            acc = part if acc is None else acc + part
        return acc

    gi = mv(0) + bias(bi_ih) + bias(bi_hh)
    gg = mv(1) + bias(bg_ih) + bias(bg_hh)
    go = mv(2) + bias(bo_ih) + bias(bo_hh)
    i = jax.nn.sigmoid(gi)
    g = jnp.tanh(gg)
    o = jax.nn.sigmoid(go)
    c = i * g
    h = o * jnp.tanh(c)
    h1_ref[...] = h
    h2_ref[...] = h
    c_ref[...] = c


def _lstm_pallas(idx, table, W_ih, b_ih1, b_hh1):
    w_spec = lambda off, ks: pl.BlockSpec(
        (T, HK), lambda j, off=off, ks=ks: (j + off, ks))
    b_spec = lambda off: pl.BlockSpec((T,), lambda j, off=off: (j + off,))
    in_specs = [
        pl.BlockSpec((WORD,), lambda j: (0,)),
        pl.BlockSpec((WORD, EMB), lambda j: (0, 0)),
    ]
    in_specs += [w_spec(off, ks)
                 for off in (0, 2 * NB, 3 * NB) for ks in range(KS)]
    in_specs += [b_spec(0), b_spec(2 * NB), b_spec(3 * NB)] * 2
    out_specs = [pl.BlockSpec((1, T), lambda j: (0, j))] * 3
    out_shape = [jax.ShapeDtypeStruct((1, H), jnp.float32)] * 3
    return pl.pallas_call(
        _lstm_body,
        grid=(NB,),
        in_specs=in_specs,
        out_specs=out_specs,
        out_shape=out_shape,
        scratch_shapes=[pltpu.VMEM((1, H), jnp.float32)],
    )(idx, table, *([W_ih] * (3 * KS)),
      b_ih1, b_ih1, b_ih1, b_hh1, b_hh1, b_hh1)


def kernel(input, table, W_ih, W_hh, b_ih, b_hh):
    del W_hh
    h1, h2, c = _lstm_pallas(input.astype(jnp.int32), table, W_ih,
                             b_ih, b_hh)
    return (h1.reshape(1, 1, H), h2.reshape(1, 1, H), c.reshape(1, 1, H))
